# baseline (speedup 1.0000x reference)
import functools

import jax
import jax.numpy as jnp
from jax import lax
from jax.experimental import pallas as pl
from jax.experimental.pallas import tpu as pltpu
from jax.experimental.pallas import tpu_sc as plsc

_BATCH = 16384
_D = 64
_CHUNK = 128


@functools.lru_cache(maxsize=None)
def _build():
    info = plsc.get_sparse_core_info()
    nw = info.num_cores * info.num_subcores
    b_per_w = _BATCH // nw
    n_chunks = b_per_w // _CHUNK
    mesh = plsc.VectorSubcoreMesh(core_axis_name="c", subcore_axis_name="s")

    @functools.partial(
        pl.kernel,
        mesh=mesh,
        out_type=jax.ShapeDtypeStruct((_BATCH, _D), jnp.float32),
        scratch_types=[
            pltpu.VMEM((n_chunks, _CHUNK), jnp.int32),
            pltpu.VMEM((b_per_w, _D), jnp.float32),
            pltpu.SemaphoreType.DMA,
        ],
        compiler_params=pltpu.CompilerParams(use_tc_tiling_on_sc=False),
    )
    def gather_kernel(idx_hbm, table_hbm, out_hbm, idx_v, rows_v, sem):
        wid = lax.axis_index("s") * info.num_cores + lax.axis_index("c")
        base = wid * b_per_w
        pltpu.sync_copy(idx_hbm.at[wid], idx_v)
        copies = [
            pltpu.async_copy(
                table_hbm.at[idx_v.at[j]],
                rows_v.at[pl.ds(j * _CHUNK, _CHUNK)],
                sem,
            )
            for j in range(n_chunks)
        ]
        for c in copies:
            c.wait()
        pltpu.sync_copy(rows_v, out_hbm.at[pl.ds(base, b_per_w)])

    return gather_kernel, nw, n_chunks


def kernel(s, Q_matrix):
    gather_kernel, nw, n_chunks = _build()
    idx = s.astype(jnp.int32).reshape(nw, n_chunks, _CHUNK)
    return gather_kernel(idx, Q_matrix)


# speedup vs baseline: 6.1276x; 6.1276x over previous
import functools

import jax
import jax.numpy as jnp
from jax import lax
from jax.experimental import pallas as pl
from jax.experimental.pallas import tpu as pltpu
from jax.experimental.pallas import tpu_sc as plsc

_N_ROWS = 1_000_000
_BATCH = 16384
_D = 64
_G = 16


@functools.lru_cache(maxsize=None)
def _build():
    info = plsc.get_sparse_core_info()
    nw = info.num_cores * info.num_subcores
    b_per_w = _BATCH // nw
    n_groups = b_per_w // _G
    mesh = plsc.VectorSubcoreMesh(core_axis_name="c", subcore_axis_name="s")

    @functools.partial(
        pl.kernel,
        mesh=mesh,
        out_type=jax.ShapeDtypeStruct((nw, 8, 8, b_per_w), jnp.float32),
        scratch_types=[
            pltpu.VMEM((b_per_w,), jnp.int32),
            pltpu.VMEM((8, 8, _G * 16), jnp.float32),
            pltpu.VMEM((8, 8, b_per_w), jnp.float32),
            pltpu.SemaphoreType.DMA,
            pltpu.SemaphoreType.DMA,
        ],
        compiler_params=pltpu.CompilerParams(needs_layout_passes=False),
    )
    def gather_kernel(idx_hbm, table_hbm, out_hbm, idx_v, blocks_v, rows_v, sem, sem2):
        wid = lax.axis_index("s") * info.num_cores + lax.axis_index("c")
        pltpu.sync_copy(idx_hbm.at[wid], idx_v)
        lanes = lax.iota(jnp.int32, 16)

        def group(g, carry):
            vec = idx_v[pl.ds(g * _G, _G)]
            aligned = vec & ~15
            lane = vec & 15
            copies = []
            for j in range(_G):
                copies.append(
                    pltpu.async_copy(
                        table_hbm.at[:, :, pl.ds(pl.multiple_of(aligned[j], 16), 16)],
                        blocks_v.at[:, :, pl.ds(j * 16, 16)],
                        sem,
                    )
                )
            for c in copies:
                c.wait()
            pos = lanes * 16 + lane
            for a in range(8):
                for b in range(8):
                    v = plsc.load_gather(
                        blocks_v,
                        [
                            jnp.full((16,), a, jnp.int32),
                            jnp.full((16,), b, jnp.int32),
                            pos,
                        ],
                    )
                    rows_v[a, b, pl.ds(g * _G, _G)] = v
            return carry

        lax.fori_loop(0, n_groups, group, 0, unroll=False)
        pltpu.async_copy(rows_v, out_hbm.at[wid], sem2).wait()

    return gather_kernel, nw, b_per_w


def kernel(s, Q_matrix):
    gather_kernel, nw, b_per_w = _build()
    idx = s.astype(jnp.int32).reshape(nw, b_per_w)
    table = jnp.transpose(Q_matrix).reshape(8, 8, _N_ROWS)
    out = gather_kernel(idx, table)
    return jnp.transpose(out, (0, 3, 1, 2)).reshape(_BATCH, _D)


# speedup vs baseline: 6.6938x; 1.0924x over previous
import functools

import jax
import jax.numpy as jnp
from jax import lax
from jax.experimental import pallas as pl
from jax.experimental.pallas import tpu as pltpu
from jax.experimental.pallas import tpu_sc as plsc

_N_ROWS = 1_000_000
_BATCH = 16384
_D = 64
_G = 16


@functools.lru_cache(maxsize=None)
def _build():
    info = plsc.get_sparse_core_info()
    nw = info.num_cores * info.num_subcores
    b_per_w = _BATCH // nw
    n_groups = b_per_w // _G
    mesh = plsc.VectorSubcoreMesh(core_axis_name="c", subcore_axis_name="s")

    @functools.partial(
        pl.kernel,
        mesh=mesh,
        out_type=jax.ShapeDtypeStruct((8, 8, _BATCH), jnp.float32),
        scratch_types=[
            pltpu.VMEM((b_per_w,), jnp.int32),
            pltpu.VMEM((8, 8, _G * 16), jnp.float32),
            pltpu.VMEM((8, 8, b_per_w), jnp.float32),
            pltpu.SemaphoreType.DMA,
            pltpu.SemaphoreType.DMA,
        ],
        compiler_params=pltpu.CompilerParams(needs_layout_passes=False),
    )
    def gather_kernel(idx_hbm, table_hbm, out_hbm, idx_v, blocks_v, rows_v, sem, sem2):
        wid = lax.axis_index("s") * info.num_cores + lax.axis_index("c")
        pltpu.sync_copy(idx_hbm.at[pl.ds(wid * b_per_w, b_per_w)], idx_v)
        lanes = lax.iota(jnp.int32, 16)

        def group(g, carry):
            vec = idx_v[pl.ds(g * _G, _G)]
            aligned = vec & ~15
            lane = vec & 15
            copies = []
            for j in range(_G):
                copies.append(
                    pltpu.async_copy(
                        table_hbm.at[:, :, pl.ds(pl.multiple_of(aligned[j], 16), 16)],
                        blocks_v.at[:, :, pl.ds(j * 16, 16)],
                        sem,
                    )
                )
            for c in copies:
                c.wait()
            pos = lanes * 16 + lane
            for a in range(8):
                for b in range(8):
                    v = plsc.load_gather(
                        blocks_v,
                        [
                            jnp.full((16,), a, jnp.int32),
                            jnp.full((16,), b, jnp.int32),
                            pos,
                        ],
                    )
                    rows_v[a, b, pl.ds(g * _G, _G)] = v
            return carry

        lax.fori_loop(0, n_groups, group, 0, unroll=False)
        pltpu.async_copy(
            rows_v,
            out_hbm.at[:, :, pl.ds(pl.multiple_of(wid * b_per_w, 128), b_per_w)],
            sem2,
        ).wait()

    return gather_kernel


def kernel(s, Q_matrix):
    gather_kernel = _build()
    idx = s.astype(jnp.int32)
    table = jnp.transpose(Q_matrix).reshape(8, 8, _N_ROWS)
    out = gather_kernel(idx, table)
    return jnp.transpose(out.reshape(_D, _BATCH))
